# baseline (speedup 1.0000x reference)
import jax
import jax.numpy as jnp
from jax import lax
from jax.experimental import pallas as pl
from jax.experimental.pallas import tpu as pltpu
from jax.experimental.pallas import tpu_sc as plsc

F_UNARY = 128
F_BIN = 16
NODES = 10000
EDGES = 320000

NCORES = 2
NSUB = 16
GROUP = 128
NGROUPS = EDGES // GROUP
GP_SUB = 160
NGROUPS_PAD = GP_SUB * NSUB
IDX_CHUNK = 16
NBLK = GP_SUB // IDX_CHUNK
ACC_ROWS = 10240
ROWS_SUB = ACC_ROWS // NSUB
B_ROWS = EDGES // (NCORES * NSUB)


def _sc_body(z_hbm, d_hbm, idx_hbm, outp_hbm,
             acc, ib0, ib1, db0, db1, sr0, sr1, si0, si1):
    c = lax.axis_index("core")
    s = lax.axis_index("subcore")
    col0 = pl.multiple_of(c * F_UNARY, F_UNARY)
    row0 = s * ROWS_SUB
    g0 = s * GP_SUB

    ibs = (ib0, ib1)
    dbs = (db0, db1)
    srs = (sr0, sr1)
    sis = (si0, si1)

    def read_slice(g_rel):
        gg = jnp.minimum(g0 + g_rel, NGROUPS - 1)
        e0 = pl.multiple_of(gg * GROUP, GROUP)
        return d_hbm.at[pl.ds(e0, GROUP), pl.ds(col0, F_UNARY)]

    def idx_slice(blk):
        return idx_hbm.at[c, s, pl.ds(blk * IDX_CHUNK, IDX_CHUNK)]

    pltpu.async_copy(idx_slice(0), ib0, si0)
    pltpu.async_copy(idx_slice(1), ib1, si1)
    pltpu.async_copy(read_slice(0), db0, sr0)
    pltpu.async_copy(read_slice(1), db1, sr1)

    pltpu.sync_copy(z_hbm, acc.at[pl.ds(row0, ROWS_SUB)])
    plsc.subcore_barrier()

    for blk in range(NBLK):
        p = blk % 2
        pltpu.make_async_copy(idx_slice(0), ibs[p], sis[p]).wait()

        @pl.loop(0, IDX_CHUNK, step=2)
        def _(j2, _blk=blk, _p=p):
            for b in (0, 1):
                g = _blk * IDX_CHUNK + j2 + b
                pltpu.make_async_copy(read_slice(0), dbs[b], srs[b]).wait()
                pltpu.sync_copy(dbs[b], acc.at[ibs[_p].at[j2 + b]], add=True)
                pltpu.async_copy(read_slice(g + 2), dbs[b], srs[b])

        if blk + 2 < NBLK:
            pltpu.async_copy(idx_slice(blk + 2), ibs[p], sis[p])

    pltpu.make_async_copy(read_slice(0), db0, sr0).wait()
    pltpu.make_async_copy(read_slice(0), db1, sr1).wait()

    plsc.subcore_barrier()
    pltpu.sync_copy(acc.at[pl.ds(row0, ROWS_SUB)],
                    outp_hbm.at[c, pl.ds(row0, ROWS_SUB)])


def _merge_body(p_ref, o_ref):
    o_ref[...] = p_ref[0] + p_ref[1]


def _b_body(d_ref, o_ref):
    o_ref[...] = d_ref[...]


def kernel(unary, binary, deltas, index1, index2):
    del unary, binary
    idx = jnp.concatenate(
        [index1.reshape(1, EDGES), index2.reshape(1, EDGES)], axis=0)
    pad = NGROUPS_PAD * GROUP - EDGES
    idx = jnp.pad(idx, ((0, 0), (0, pad)), constant_values=NODES)
    idx = idx.reshape(NCORES, NSUB, GP_SUB, GROUP)
    zeros = jnp.zeros((ROWS_SUB, F_UNARY), jnp.float32)

    mesh = plsc.VectorSubcoreMesh(core_axis_name="core",
                                  subcore_axis_name="subcore")
    sc_fn = pl.kernel(
        _sc_body,
        out_type=jax.ShapeDtypeStruct((NCORES, ACC_ROWS, F_UNARY),
                                      jnp.float32),
        mesh=mesh,
        scratch_types=[
            pltpu.VMEM_SHARED((ACC_ROWS, F_UNARY), jnp.float32),
            pltpu.VMEM((IDX_CHUNK, GROUP), jnp.int32),
            pltpu.VMEM((IDX_CHUNK, GROUP), jnp.int32),
            pltpu.VMEM((GROUP, F_UNARY), jnp.float32),
            pltpu.VMEM((GROUP, F_UNARY), jnp.float32),
            pltpu.SemaphoreType.DMA,
            pltpu.SemaphoreType.DMA,
            pltpu.SemaphoreType.DMA,
            pltpu.SemaphoreType.DMA,
        ],
    )
    outp = sc_fn(zeros, deltas, idx)

    b = deltas[:, 2 * F_UNARY:]

    merge = pl.pallas_call(
        _merge_body,
        grid=(NODES // 400,),
        in_specs=[pl.BlockSpec((NCORES, 400, F_UNARY), lambda i: (0, i, 0))],
        out_specs=pl.BlockSpec((400, F_UNARY), lambda i: (i, 0)),
        out_shape=jax.ShapeDtypeStruct((NODES, F_UNARY), jnp.float32),
    )
    out = merge(outp)
    return (out, b)


# speedup vs baseline: 1.2668x; 1.2668x over previous
import jax
import jax.numpy as jnp
from jax import lax
from jax.experimental import pallas as pl
from jax.experimental.pallas import tpu as pltpu
from jax.experimental.pallas import tpu_sc as plsc

F_UNARY = 128
F_BIN = 16
NODES = 10000
EDGES = 320000

NCORES = 2
NSUB = 16
GROUP = 128
NGROUPS = EDGES // GROUP

NCHUNK = 4
CH_GROUPS = NGROUPS // NCHUNK
CH_EDGES = CH_GROUPS * GROUP
GP_SUB = 40
ACC_ROWS = 10240
ROWS_SUB = ACC_ROWS // NSUB

E_BLK = 3200
B_BLK = 16000


def _sc_body(z_hbm, d_hbm, idx_hbm, outp_hbm,
             acc, idx_v, db0, db1, sr0, sr1):
    c = lax.axis_index("core")
    s = lax.axis_index("subcore")
    row0 = s * ROWS_SUB
    g0 = s * GP_SUB

    dbs = (db0, db1)
    srs = (sr0, sr1)

    def read_slice(g_rel):
        gg = jnp.minimum(g0 + g_rel, CH_GROUPS - 1)
        e0 = pl.multiple_of(gg * GROUP, GROUP)
        return d_hbm.at[c, pl.ds(e0, GROUP)]

    pltpu.async_copy(read_slice(0), db0, sr0)
    pltpu.async_copy(read_slice(1), db1, sr1)
    pltpu.sync_copy(idx_hbm.at[c, s], idx_v)

    pltpu.sync_copy(z_hbm, acc.at[pl.ds(row0, ROWS_SUB)])
    plsc.subcore_barrier()

    @pl.loop(0, GP_SUB, step=2)
    def _(j2):
        for b in (0, 1):
            pltpu.make_async_copy(read_slice(0), dbs[b], srs[b]).wait()
            pltpu.sync_copy(dbs[b], acc.at[idx_v.at[j2 + b]], add=True)
            pltpu.async_copy(read_slice(j2 + b + 2), dbs[b], srs[b])

    pltpu.make_async_copy(read_slice(0), db0, sr0).wait()
    pltpu.make_async_copy(read_slice(0), db1, sr1).wait()

    plsc.subcore_barrier()
    pltpu.sync_copy(acc.at[pl.ds(row0, ROWS_SUB)],
                    outp_hbm.at[c, pl.ds(row0, ROWS_SUB)])


def _merge_body(p0, p1, p2, p3, o_ref):
    acc = p0[0] + p0[1]
    for p in (p1, p2, p3):
        acc = acc + p[0] + p[1]
    o_ref[...] = acc


def _tr_body(d_ref, u_ref):
    u_ref[0] = lax.transpose(d_ref[pl.ds(0, F_UNARY), :], (1, 0))
    u_ref[1] = lax.transpose(d_ref[pl.ds(F_UNARY, F_UNARY), :], (1, 0))


def _b_body(d_ref, o_ref):
    o_ref[...] = d_ref[...]


def kernel(unary, binary, deltas, index1, index2):
    del unary, binary
    d_t = jnp.transpose(deltas)

    idx = jnp.concatenate(
        [index1.reshape(1, EDGES), index2.reshape(1, EDGES)], axis=0)
    idx = idx.reshape(NCORES, NCHUNK, CH_GROUPS, GROUP)
    idx = jnp.pad(idx, ((0, 0), (0, 0), (0, NSUB * GP_SUB - CH_GROUPS),
                        (0, 0)), constant_values=NODES)
    idx = idx.reshape(NCORES, NCHUNK, NSUB, GP_SUB, GROUP)
    idx = jnp.transpose(idx, (1, 0, 2, 3, 4))

    zeros = jnp.zeros((ROWS_SUB, F_UNARY), jnp.float32)
    mesh = plsc.VectorSubcoreMesh(core_axis_name="core",
                                  subcore_axis_name="subcore")
    sc_fn = pl.kernel(
        _sc_body,
        out_type=jax.ShapeDtypeStruct((NCORES, ACC_ROWS, F_UNARY),
                                      jnp.float32),
        mesh=mesh,
        scratch_types=[
            pltpu.VMEM_SHARED((ACC_ROWS, F_UNARY), jnp.float32),
            pltpu.VMEM((GP_SUB, GROUP), jnp.int32),
            pltpu.VMEM((GROUP, F_UNARY), jnp.float32),
            pltpu.VMEM((GROUP, F_UNARY), jnp.float32),
            pltpu.SemaphoreType.DMA,
            pltpu.SemaphoreType.DMA,
        ],
    )

    partials = []
    for k in range(NCHUNK):
        tr = pl.pallas_call(
            _tr_body,
            grid=(CH_EDGES // E_BLK,),
            in_specs=[pl.BlockSpec((2 * F_UNARY + F_BIN, E_BLK),
                                   lambda i, _k=k: (0, i + _k * (CH_EDGES // E_BLK)))],
            out_specs=pl.BlockSpec((NCORES, E_BLK, F_UNARY),
                                   lambda i: (0, i, 0)),
            out_shape=jax.ShapeDtypeStruct((NCORES, CH_EDGES, F_UNARY),
                                           jnp.float32),
        )
        uxy_k = tr(d_t)
        partials.append(sc_fn(zeros, uxy_k, idx[k]))

    bcopy = pl.pallas_call(
        _b_body,
        grid=(EDGES // B_BLK,),
        in_specs=[pl.BlockSpec((F_BIN, B_BLK), lambda i: (16, i))],
        out_specs=pl.BlockSpec((F_BIN, B_BLK), lambda i: (0, i)),
        out_shape=jax.ShapeDtypeStruct((F_BIN, EDGES), jnp.float32),
    )
    b = jnp.transpose(bcopy(d_t))

    merge = pl.pallas_call(
        _merge_body,
        grid=(NODES // 400,),
        in_specs=[pl.BlockSpec((NCORES, 400, F_UNARY),
                               lambda i: (0, i, 0))] * NCHUNK,
        out_specs=pl.BlockSpec((400, F_UNARY), lambda i: (i, 0)),
        out_shape=jax.ShapeDtypeStruct((NODES, F_UNARY), jnp.float32),
    )
    out = merge(*partials)
    return (out, b)


# speedup vs baseline: 1.2732x; 1.0051x over previous
import jax
import jax.numpy as jnp
from jax import lax
from jax.experimental import pallas as pl
from jax.experimental.pallas import tpu as pltpu
from jax.experimental.pallas import tpu_sc as plsc

F_UNARY = 128
F_BIN = 16
NODES = 10000
EDGES = 320000

NCORES = 2
NSUB = 16
GROUP = 128
NGROUPS = EDGES // GROUP

CH_REAL = (260, 1120, 1120)
CH_START = (0, 260, 1380)
CH_GP = (17, 70, 70)
NCHUNK = 3

ACC_ROWS = 10240
ROWS_SUB = ACC_ROWS // NSUB

E_BLK = 2560
B_BLK = 16000


def _make_sc_body(ch_real, ch_gp):
    def _sc_body(z_hbm, d_hbm, idx_hbm, outp_hbm,
                 acc, idx_v, db0, db1, sr0, sr1):
        c = lax.axis_index("core")
        s = lax.axis_index("subcore")
        row0 = s * ROWS_SUB
        g0 = s * ch_gp

        dbs = (db0, db1)
        srs = (sr0, sr1)

        def read_slice(g_rel):
            gg = jnp.minimum(g0 + g_rel, ch_real - 1)
            e0 = pl.multiple_of(gg * GROUP, GROUP)
            return d_hbm.at[c, pl.ds(e0, GROUP)]

        pltpu.async_copy(read_slice(0), db0, sr0)
        pltpu.async_copy(read_slice(1), db1, sr1)
        pltpu.sync_copy(idx_hbm.at[c, s], idx_v)

        pltpu.sync_copy(z_hbm, acc.at[pl.ds(row0, ROWS_SUB)])
        plsc.subcore_barrier()

        @pl.loop(0, ch_gp, step=2)
        def _(j2):
            for b in (0, 1):
                pltpu.make_async_copy(read_slice(0), dbs[b], srs[b]).wait()
                pltpu.sync_copy(dbs[b], acc.at[idx_v.at[j2 + b]], add=True)
                pltpu.async_copy(read_slice(j2 + b + 2), dbs[b], srs[b])

        pltpu.make_async_copy(read_slice(0), db0, sr0).wait()
        pltpu.make_async_copy(read_slice(0), db1, sr1).wait()

        plsc.subcore_barrier()
        pltpu.sync_copy(acc.at[pl.ds(row0, ROWS_SUB)],
                        outp_hbm.at[c, pl.ds(row0, ROWS_SUB)])

    return _sc_body


def _merge_body(p0, p1, p2, o_ref):
    o_ref[...] = (p0[0] + p0[1]) + (p1[0] + p1[1]) + (p2[0] + p2[1])


def _tr_body(d_ref, u_ref):
    u_ref[0] = lax.transpose(d_ref[pl.ds(0, F_UNARY), :], (1, 0))
    u_ref[1] = lax.transpose(d_ref[pl.ds(F_UNARY, F_UNARY), :], (1, 0))


def _b_body(d_ref, o_ref):
    o_ref[...] = d_ref[...]


def kernel(unary, binary, deltas, index1, index2):
    del unary, binary
    d_t = jnp.transpose(deltas)

    idx_all = jnp.concatenate(
        [index1.reshape(1, NGROUPS, GROUP), index2.reshape(1, NGROUPS, GROUP)],
        axis=0)

    zeros = jnp.zeros((ROWS_SUB, F_UNARY), jnp.float32)
    mesh = plsc.VectorSubcoreMesh(core_axis_name="core",
                                  subcore_axis_name="subcore")

    partials = []
    for k in range(NCHUNK):
        real, start, gp = CH_REAL[k], CH_START[k], CH_GP[k]
        blk0 = start * GROUP // E_BLK
        nblk = real * GROUP // E_BLK
        tr = pl.pallas_call(
            _tr_body,
            grid=(nblk,),
            in_specs=[pl.BlockSpec((2 * F_UNARY + F_BIN, E_BLK),
                                   lambda i, _b=blk0: (0, i + _b))],
            out_specs=pl.BlockSpec((NCORES, E_BLK, F_UNARY),
                                   lambda i: (0, i, 0)),
            out_shape=jax.ShapeDtypeStruct((NCORES, real * GROUP, F_UNARY),
                                           jnp.float32),
        )
        uxy_k = tr(d_t)

        idx_k = idx_all[:, start:start + real]
        idx_k = jnp.pad(idx_k, ((0, 0), (0, NSUB * gp - real), (0, 0)),
                        constant_values=NODES)
        idx_k = idx_k.reshape(NCORES, NSUB, gp, GROUP)

        sc_fn = pl.kernel(
            _make_sc_body(real, gp),
            out_type=jax.ShapeDtypeStruct((NCORES, ACC_ROWS, F_UNARY),
                                          jnp.float32),
            mesh=mesh,
            scratch_types=[
                pltpu.VMEM_SHARED((ACC_ROWS, F_UNARY), jnp.float32),
                pltpu.VMEM((gp, GROUP), jnp.int32),
                pltpu.VMEM((GROUP, F_UNARY), jnp.float32),
                pltpu.VMEM((GROUP, F_UNARY), jnp.float32),
                pltpu.SemaphoreType.DMA,
                pltpu.SemaphoreType.DMA,
            ],
        )
        partials.append(sc_fn(zeros, uxy_k, idx_k))

    bcopy = pl.pallas_call(
        _b_body,
        grid=(EDGES // B_BLK,),
        in_specs=[pl.BlockSpec((F_BIN, B_BLK), lambda i: (16, i))],
        out_specs=pl.BlockSpec((F_BIN, B_BLK), lambda i: (0, i)),
        out_shape=jax.ShapeDtypeStruct((F_BIN, EDGES), jnp.float32),
    )
    b = jnp.transpose(bcopy(d_t))

    merge = pl.pallas_call(
        _merge_body,
        grid=(NODES // 400,),
        in_specs=[pl.BlockSpec((NCORES, 400, F_UNARY),
                               lambda i: (0, i, 0))] * NCHUNK,
        out_specs=pl.BlockSpec((400, F_UNARY), lambda i: (i, 0)),
        out_shape=jax.ShapeDtypeStruct((NODES, F_UNARY), jnp.float32),
    )
    out = merge(*partials)
    return (out, b)


# speedup vs baseline: 1.2924x; 1.0150x over previous
import jax
import jax.numpy as jnp
from jax import lax
from jax.experimental import pallas as pl
from jax.experimental.pallas import tpu as pltpu
from jax.experimental.pallas import tpu_sc as plsc

F_UNARY = 128
F_BIN = 16
NODES = 10000
EDGES = 320000

NCORES = 2
NSUB = 16
GROUP = 128
NGROUPS = EDGES // GROUP
GP_SUB = 160
NGROUPS_PAD = GP_SUB * NSUB
IDX_CHUNK = 16
NBLK = GP_SUB // IDX_CHUNK
ACC_ROWS = 10240
ROWS_SUB = ACC_ROWS // NSUB
B_ROWS = EDGES // (NCORES * NSUB)


def _sc_body(z_hbm, d_hbm, idx_hbm, outp_hbm,
             acc, ib0, ib1, db0, db1, sr0, sr1, si0, si1):
    c = lax.axis_index("core")
    s = lax.axis_index("subcore")
    row0 = s * ROWS_SUB
    g0 = s * GP_SUB

    ibs = (ib0, ib1)
    dbs = (db0, db1)
    srs = (sr0, sr1)
    sis = (si0, si1)

    def read_slice(g_rel):
        gg = jnp.minimum(g0 + g_rel, NGROUPS - 1)
        e0 = pl.multiple_of(gg * GROUP, GROUP)
        return d_hbm.at[c, pl.ds(e0, GROUP)]

    def idx_slice(blk):
        return idx_hbm.at[c, s, pl.ds(blk * IDX_CHUNK, IDX_CHUNK)]

    pltpu.async_copy(idx_slice(0), ib0, si0)
    pltpu.async_copy(idx_slice(1), ib1, si1)
    pltpu.async_copy(read_slice(0), db0, sr0)
    pltpu.async_copy(read_slice(1), db1, sr1)

    pltpu.sync_copy(z_hbm, acc.at[pl.ds(row0, ROWS_SUB)])
    plsc.subcore_barrier()

    for blk in range(NBLK):
        p = blk % 2
        pltpu.make_async_copy(idx_slice(0), ibs[p], sis[p]).wait()

        @pl.loop(0, IDX_CHUNK, step=2)
        def _(j2, _blk=blk, _p=p):
            for b in (0, 1):
                g = _blk * IDX_CHUNK + j2 + b
                pltpu.make_async_copy(read_slice(0), dbs[b], srs[b]).wait()
                pltpu.sync_copy(dbs[b], acc.at[ibs[_p].at[j2 + b]], add=True)
                pltpu.async_copy(read_slice(g + 2), dbs[b], srs[b])

        if blk + 2 < NBLK:
            pltpu.async_copy(idx_slice(blk + 2), ibs[p], sis[p])

    pltpu.make_async_copy(read_slice(0), db0, sr0).wait()
    pltpu.make_async_copy(read_slice(0), db1, sr1).wait()

    plsc.subcore_barrier()
    pltpu.sync_copy(acc.at[pl.ds(row0, ROWS_SUB)],
                    outp_hbm.at[c, pl.ds(row0, ROWS_SUB)])


def _merge_body(p_ref, o_ref):
    o_ref[...] = p_ref[0] + p_ref[1]


E_BLK = 2560


def _tr_body(d_ref, u_ref, b_ref):
    u_ref[0] = lax.transpose(d_ref[pl.ds(0, F_UNARY), :], (1, 0))
    u_ref[1] = lax.transpose(d_ref[pl.ds(F_UNARY, F_UNARY), :], (1, 0))
    b_ref[...] = d_ref[pl.ds(2 * F_UNARY, F_BIN), :]


def kernel(unary, binary, deltas, index1, index2):
    del unary, binary
    d_t = jnp.transpose(deltas)
    tr = pl.pallas_call(
        _tr_body,
        grid=(EDGES // E_BLK,),
        in_specs=[pl.BlockSpec((2 * F_UNARY + F_BIN, E_BLK),
                               lambda i: (0, i))],
        out_specs=(pl.BlockSpec((NCORES, E_BLK, F_UNARY),
                                lambda i: (0, i, 0)),
                   pl.BlockSpec((F_BIN, E_BLK), lambda i: (0, i))),
        out_shape=(jax.ShapeDtypeStruct((NCORES, EDGES, F_UNARY),
                                        jnp.float32),
                   jax.ShapeDtypeStruct((F_BIN, EDGES), jnp.float32)),
    )
    uxy, b_t = tr(d_t)
    b = jnp.transpose(b_t)

    idx = jnp.concatenate(
        [index1.reshape(1, EDGES), index2.reshape(1, EDGES)], axis=0)
    pad = NGROUPS_PAD * GROUP - EDGES
    idx = jnp.pad(idx, ((0, 0), (0, pad)), constant_values=NODES)
    idx = idx.reshape(NCORES, NSUB, GP_SUB, GROUP)
    zeros = jnp.zeros((ROWS_SUB, F_UNARY), jnp.float32)

    mesh = plsc.VectorSubcoreMesh(core_axis_name="core",
                                  subcore_axis_name="subcore")
    sc_fn = pl.kernel(
        _sc_body,
        out_type=jax.ShapeDtypeStruct((NCORES, ACC_ROWS, F_UNARY),
                                      jnp.float32),
        mesh=mesh,
        scratch_types=[
            pltpu.VMEM_SHARED((ACC_ROWS, F_UNARY), jnp.float32),
            pltpu.VMEM((IDX_CHUNK, GROUP), jnp.int32),
            pltpu.VMEM((IDX_CHUNK, GROUP), jnp.int32),
            pltpu.VMEM((GROUP, F_UNARY), jnp.float32),
            pltpu.VMEM((GROUP, F_UNARY), jnp.float32),
            pltpu.SemaphoreType.DMA,
            pltpu.SemaphoreType.DMA,
            pltpu.SemaphoreType.DMA,
            pltpu.SemaphoreType.DMA,
        ],
    )
    outp = sc_fn(zeros, uxy, idx)

    merge = pl.pallas_call(
        _merge_body,
        grid=(NODES // 400,),
        in_specs=[pl.BlockSpec((NCORES, 400, F_UNARY), lambda i: (0, i, 0))],
        out_specs=pl.BlockSpec((400, F_UNARY), lambda i: (i, 0)),
        out_shape=jax.ShapeDtypeStruct((NODES, F_UNARY), jnp.float32),
    )
    out = merge(outp)
    return (out, b)
